# Initial kernel scaffold; baseline (speedup 1.0000x reference)
import jax
import jax.numpy as jnp
from jax.experimental import pallas as pl


def kernel(loc_preds, loc_targets, conf_preds, label_targets):
    raise NotImplementedError("write your pallas kernel here")


# speedup vs baseline: 1.3427x; 1.3427x over previous
import functools

import jax
import jax.numpy as jnp
from jax import lax
from jax.experimental import pallas as pl

_N = 128
_NB = 8732
_NC = 21
_CHUNK = 1024
_NCHUNK = 9
_PADW = _CHUNK * _NCHUNK
_PAD_NEG = -3.0e38

_I32_MIN = jnp.iinfo(jnp.int32).min


def _stage_a_body(conf_ref, lab_ref, lp_ref, lt_ref,
                  cl_ref, accloc_ref, accnll_ref, accnp_ref):
    r = pl.program_id(0)
    c = pl.program_id(1)

    zero = jnp.zeros((1, 1), jnp.float32)

    @pl.when((r == 0) & (c == 0))
    def _init():
        accloc_ref[...] = zero
        accnll_ref[...] = zero
        accnp_ref[...] = zero

    x = conf_ref[0]
    lab = lab_ref[0]

    rowid = lax.broadcasted_iota(jnp.int32, (_CHUNK, 1), 0) + c * _CHUNK
    valid = rowid < _NB
    pos = (lab > 0) & valid

    m = jnp.max(x, axis=1, keepdims=True)
    e = jnp.exp(x - m)
    s = jnp.sum(e, axis=1, keepdims=True)
    lse = jnp.log(s) + m
    cls_iota = lax.broadcasted_iota(jnp.int32, (_CHUNK, _NC), 1)
    pick = jnp.sum(jnp.where(cls_iota == lab, x, 0.0), axis=1, keepdims=True)
    nll = lse - pick

    cl = jnp.where(valid, jnp.where(pos, -1.0, nll), _PAD_NEG)
    cl_ref[0] = cl

    accnll_ref[...] += jnp.sum(jnp.where(pos, nll, 0.0)).reshape(1, 1)
    accnp_ref[...] += jnp.sum(jnp.where(pos, 1.0, 0.0)).reshape(1, 1)

    d = lp_ref[0] - lt_ref[0]
    ad = jnp.abs(d)
    sl1 = jnp.where(ad < 1.0, 0.5 * d * d, ad - 0.5)
    accloc_ref[...] += jnp.sum(jnp.where(pos, sl1, 0.0)).reshape(1, 1)


def _stage_b_body(cl_ref, out_ref, *, rows):
    pid = pl.program_id(0)
    x = cl_ref[...]
    i = lax.bitcast_convert_type(x, jnp.int32)
    kb = jnp.where(i >= 0, i, i ^ 0x7FFFFFFF)

    p = jnp.sum((x == -1.0).astype(jnp.int32), axis=1, keepdims=True)
    j = jnp.minimum(jnp.minimum(3 * p, _NB - 1), _NB - p)

    def bit_step(it, prefix):
        t = prefix + (jnp.int32(1) << (31 - it))
        cnt = jnp.sum((kb >= t).astype(jnp.int32), axis=1, keepdims=True)
        return jnp.where(cnt >= j, t, prefix)

    prefix = lax.fori_loop(
        0, 32, bit_step, jnp.full((rows, 1), _I32_MIN, jnp.int32))

    gt = kb > prefix
    c_gt = jnp.sum(gt.astype(jnp.int32), axis=1, keepdims=True)
    sum_gt = jnp.sum(jnp.where(gt, x, 0.0), axis=1, keepdims=True)
    tbits = jnp.where(prefix >= 0, prefix, prefix ^ 0x7FFFFFFF)
    tval = lax.bitcast_convert_type(tbits, jnp.float32)
    row = jnp.where(j > 0, sum_gt + (j - c_gt).astype(jnp.float32) * tval, 0.0)

    @pl.when(pid == 0)
    def _init():
        out_ref[...] = jnp.zeros((1, 1), jnp.float32)

    out_ref[...] += jnp.sum(row).reshape(1, 1)


def kernel(loc_preds, loc_targets, conf_preds, label_targets):
    labels = label_targets.astype(jnp.int32).reshape(_N, _NB, 1)

    cl, loc_loss, nll_pos, num_pos = pl.pallas_call(
        _stage_a_body,
        grid=(_N, _NCHUNK),
        in_specs=[
            pl.BlockSpec((1, _CHUNK, _NC), lambda r, c: (r, c, 0)),
            pl.BlockSpec((1, _CHUNK, 1), lambda r, c: (r, c, 0)),
            pl.BlockSpec((1, _CHUNK, 4), lambda r, c: (r, c, 0)),
            pl.BlockSpec((1, _CHUNK, 4), lambda r, c: (r, c, 0)),
        ],
        out_specs=[
            pl.BlockSpec((1, _CHUNK, 1), lambda r, c: (r, c, 0)),
            pl.BlockSpec((1, 1), lambda r, c: (0, 0)),
            pl.BlockSpec((1, 1), lambda r, c: (0, 0)),
            pl.BlockSpec((1, 1), lambda r, c: (0, 0)),
        ],
        out_shape=[
            jax.ShapeDtypeStruct((_N, _PADW, 1), jnp.float32),
            jax.ShapeDtypeStruct((1, 1), jnp.float32),
            jax.ShapeDtypeStruct((1, 1), jnp.float32),
            jax.ShapeDtypeStruct((1, 1), jnp.float32),
        ],
    )(conf_preds, labels, loc_preds, loc_targets)

    rows = 16
    conf_neg = pl.pallas_call(
        functools.partial(_stage_b_body, rows=rows),
        grid=(_N // rows,),
        in_specs=[pl.BlockSpec((rows, _PADW), lambda g: (g, 0))],
        out_specs=pl.BlockSpec((1, 1), lambda g: (0, 0)),
        out_shape=jax.ShapeDtypeStruct((1, 1), jnp.float32),
    )(cl.reshape(_N, _PADW))

    nm = num_pos[0, 0]
    total = (loc_loss[0, 0] + nll_pos[0, 0] + conf_neg[0, 0]) / nm
    return jnp.where(nm == 0.0, 0.0, total)
